# baseline (speedup 1.0000x reference)
import jax
import jax.numpy as jnp
from jax import lax
from jax.experimental import pallas as pl
from jax.experimental.pallas import tpu as pltpu

B, S, H = 2, 2048, 2048
E, TOP_K, R = 8, 2, 64
R2 = TOP_K * R

S_TILE = 1024


def _body(x_ref, rw_ref, wd_any, wu_any, out_ref,
          wdc, wuc, gvs, wd0f, wd1f, wu0f, wu1f, sem):
    s = pl.program_id(1)
    col = lax.broadcasted_iota(jnp.int32, (1, R2), 1)

    @pl.when(s == 0)
    def _():
        cls = x_ref[0, 0:1, :].astype(jnp.bfloat16).astype(jnp.float32)
        rw = rw_ref[...].astype(jnp.bfloat16).astype(jnp.float32)
        lv = jnp.sum(cls * rw, axis=1, keepdims=True)
        erow = lax.broadcasted_iota(jnp.int32, (E, 1), 0)
        m1 = jnp.max(lv)
        i1 = jnp.min(jnp.where(lv == m1, erow, E))
        masked = jnp.where(erow == i1, -jnp.inf, lv)
        m2 = jnp.max(masked)
        i2 = jnp.min(jnp.where(masked == m2, erow, E))
        ev = jnp.exp(jnp.full((1, R2), m2 - m1, jnp.float32))
        gvs[...] = jnp.where(col < R, 1.0, ev) / (1.0 + ev)

        c0 = pltpu.make_async_copy(wd_any.at[i1], wd0f, sem)
        c0.start()
        c1 = pltpu.make_async_copy(wd_any.at[i2], wd1f, sem)
        c1.start()
        c2 = pltpu.make_async_copy(wu_any.at[i1], wu0f, sem)
        c2.start()
        c3 = pltpu.make_async_copy(wu_any.at[i2], wu1f, sem)
        c3.start()
        c0.wait()
        c1.wait()
        c2.wait()
        c3.wait()
        wdc[:R, :] = wd0f[...].astype(jnp.bfloat16)
        wdc[R:, :] = wd1f[...].astype(jnp.bfloat16)
        wuc[:, :R] = wu0f[...].astype(jnp.bfloat16)
        wuc[:, R:] = wu1f[...].astype(jnp.bfloat16)

    nt = (((1,), (1,)), ((), ()))
    xb = x_ref[0]
    h = lax.dot_general(xb.astype(jnp.bfloat16), wdc[...], nt,
                        preferred_element_type=jnp.float32)
    hg = (h * gvs[...]).astype(jnp.bfloat16)
    eo = lax.dot_general(hg, wuc[...], nt,
                         preferred_element_type=jnp.float32)
    out_ref[0] = xb + eo


@jax.jit
def _moe_call(x, router_w, Wd, Wu):
    grid = (B, S // S_TILE)
    return pl.pallas_call(
        _body,
        grid=grid,
        in_specs=[
            pl.BlockSpec((1, S_TILE, H), lambda b, s: (b, s, 0)),
            pl.BlockSpec((E, H), lambda b, s: (0, 0)),
            pl.BlockSpec(memory_space=pl.ANY),
            pl.BlockSpec(memory_space=pl.ANY),
        ],
        out_specs=pl.BlockSpec((1, S_TILE, H), lambda b, s: (b, s, 0)),
        scratch_shapes=[
            pltpu.VMEM((R2, H), jnp.bfloat16),
            pltpu.VMEM((H, R2), jnp.bfloat16),
            pltpu.VMEM((1, R2), jnp.float32),
            pltpu.VMEM((R, H), jnp.float32),
            pltpu.VMEM((R, H), jnp.float32),
            pltpu.VMEM((H, R), jnp.float32),
            pltpu.VMEM((H, R), jnp.float32),
            pltpu.SemaphoreType.DMA,
        ],
        out_shape=jax.ShapeDtypeStruct((B, S, H), jnp.float32),
        compiler_params=pltpu.CompilerParams(
            dimension_semantics=("arbitrary", "arbitrary"),
        ),
    )(x, router_w, Wd, Wu)


def kernel(x, router_w, Wd, Wu):
    return _moe_call(x, router_w, Wd, Wu)
